# baseline (speedup 1.0000x reference)
import jax
import jax.numpy as jnp
from jax import lax
from jax.experimental import pallas as pl
from jax.experimental.pallas import tpu as pltpu
from jax.experimental.pallas import tpu_sc as plsc

N_NODES = 10000
N_EDGES = 320000
D_FEAT = 128

NC = 2
NS = 16
NW = NC * NS
EDGES_PER_W = N_EDGES // NW
CHUNK = 128
NCHUNK = EDGES_PER_W // CHUNK
TAIL = EDGES_PER_W - NCHUNK * CHUNK
NBLK = 13
BLKCH = NCHUNK // NBLK
BLKE = BLKCH * CHUNK
ROWS_PER_TILE = N_NODES // NS


def _sc_body(ei_hbm, x_hbm, za_hbm, zc_hbm, outa_hbm, outc_hbm,
             src_a, src_b, dst_a, dst_b, src_t, dst_t, rows0, rows1, c_tile,
             a_sh, sem0, sem1, semi):
    c = lax.axis_index("c")
    s = lax.axis_index("s")
    wid = s * NC + c
    lo = s * ROWS_PER_TILE
    ebase = wid * EDGES_PER_W

    pltpu.sync_copy(za_hbm.at[pl.ds(lo, ROWS_PER_TILE)],
                    a_sh.at[pl.ds(lo, ROWS_PER_TILE)])
    pltpu.sync_copy(zc_hbm, c_tile)
    pltpu.sync_copy(ei_hbm.at[0].at[pl.ds(ebase, BLKE)], src_a)
    pltpu.sync_copy(ei_hbm.at[1].at[pl.ds(ebase, BLKE)], dst_a)
    pltpu.sync_copy(ei_hbm.at[0].at[pl.ds(ebase + NCHUNK * CHUNK, TAIL)], src_t)
    pltpu.sync_copy(ei_hbm.at[1].at[pl.ds(ebase + NCHUNK * CHUNK, TAIL)], dst_t)
    plsc.subcore_barrier()

    ones16 = jnp.ones((16,), jnp.float32)

    def gather(sv, j, rows, sem):
        return pltpu.async_copy(
            x_hbm.at[sv.at[pl.ds(j * CHUNK, CHUNK)]], rows, sem)

    def drain(rows, sem):
        pltpu.make_async_copy(x_hbm.at[src_a.at[pl.ds(0, CHUNK)]], rows,
                              sem).wait()

    def counts(dv, j):
        for k in range(CHUNK // 16):
            idx = dv[pl.ds(j * CHUNK + k * 16, 16)]
            plsc.addupdate_scatter(c_tile, [idx], ones16)

    def scatter(dv, j, rows):
        pltpu.sync_copy(rows, a_sh.at[dv.at[pl.ds(j * CHUNK, CHUNK)]],
                        add=True)

    idx_bufs = (src_a, dst_a), (src_b, dst_b)
    for blk in range(NBLK):
        sv, dv = idx_bufs[blk % 2]
        nsv, ndv = idx_bufs[(blk + 1) % 2]
        if blk + 1 < NBLK:
            nbase = ebase + (blk + 1) * BLKE
            pltpu.async_copy(ei_hbm.at[0].at[pl.ds(nbase, BLKE)], nsv, semi)
            pltpu.async_copy(ei_hbm.at[1].at[pl.ds(nbase, BLKE)], ndv, semi)

        gather(sv, 0, rows0, sem0)

        def pair_body(i, carry, sv=sv, dv=dv):
            j = 2 * i
            gather(sv, j + 1, rows1, sem1)
            counts(dv, j)
            drain(rows0, sem0)
            scatter(dv, j, rows0)
            gather(sv, j + 2, rows0, sem0)
            counts(dv, j + 1)
            drain(rows1, sem1)
            scatter(dv, j + 1, rows1)
            return carry

        lax.fori_loop(0, BLKCH // 2 - 1, pair_body, 0)
        gather(sv, BLKCH - 1, rows1, sem1)
        counts(dv, BLKCH - 2)
        drain(rows0, sem0)
        scatter(dv, BLKCH - 2, rows0)
        counts(dv, BLKCH - 1)
        drain(rows1, sem1)
        scatter(dv, BLKCH - 1, rows1)

        if blk + 1 < NBLK:
            pltpu.make_async_copy(ei_hbm.at[0].at[pl.ds(0, BLKE)], nsv,
                                  semi).wait()
            pltpu.make_async_copy(ei_hbm.at[1].at[pl.ds(0, BLKE)], ndv,
                                  semi).wait()

    pltpu.async_copy(x_hbm.at[src_t], rows0.at[pl.ds(0, TAIL)], sem0).wait()
    plsc.addupdate_scatter(c_tile, [dst_t[...]], ones16)
    pltpu.sync_copy(rows0.at[pl.ds(0, TAIL)], a_sh.at[dst_t], add=True)
    plsc.subcore_barrier()

    pltpu.sync_copy(a_sh.at[pl.ds(lo, ROWS_PER_TILE)],
                    outa_hbm.at[c].at[pl.ds(lo, ROWS_PER_TILE)])
    pltpu.sync_copy(c_tile, outc_hbm.at[c].at[s])


def _sc_aggregate(edge_index, x, zeros_a, zeros_c):
    mesh = plsc.VectorSubcoreMesh(core_axis_name="c", subcore_axis_name="s",
                                  num_cores=NC, num_subcores=NS)
    return pl.kernel(
        _sc_body,
        out_type=(
            jax.ShapeDtypeStruct((NC, N_NODES, D_FEAT), jnp.float32),
            jax.ShapeDtypeStruct((NC, NS, N_NODES), jnp.float32),
        ),
        mesh=mesh,
        compiler_params=pltpu.CompilerParams(use_tc_tiling_on_sc=False,
                                             needs_layout_passes=False),
        scratch_types=[
            pltpu.VMEM((BLKE,), jnp.int32),
            pltpu.VMEM((BLKE,), jnp.int32),
            pltpu.VMEM((BLKE,), jnp.int32),
            pltpu.VMEM((BLKE,), jnp.int32),
            pltpu.VMEM((TAIL,), jnp.int32),
            pltpu.VMEM((TAIL,), jnp.int32),
            pltpu.VMEM((CHUNK, D_FEAT), jnp.float32),
            pltpu.VMEM((CHUNK, D_FEAT), jnp.float32),
            pltpu.VMEM((N_NODES,), jnp.float32),
            pltpu.VMEM_SHARED((N_NODES, D_FEAT), jnp.float32),
            pltpu.SemaphoreType.DMA,
            pltpu.SemaphoreType.DMA,
            pltpu.SemaphoreType.DMA,
        ],
    )(edge_index, x, zeros_a, zeros_c)


MM_BLK = 1000


def _mm_body(a_ref, c_ref, w_ref, b_ref, o_ref):
    a = a_ref[0] + a_ref[1]
    cnt = jnp.sum(c_ref[...], axis=1)
    o_ref[...] = (jnp.dot(a, w_ref[...], preferred_element_type=jnp.float32)
                  + cnt[:, None] * b_ref[...])


def _tc_finish(parts_a, parts_ct, W, b2d):
    return pl.pallas_call(
        _mm_body,
        grid=(N_NODES // MM_BLK,),
        in_specs=[
            pl.BlockSpec((NC, MM_BLK, D_FEAT), lambda i: (0, i, 0)),
            pl.BlockSpec((MM_BLK, NC * NS), lambda i: (i, 0)),
            pl.BlockSpec((D_FEAT, D_FEAT), lambda i: (0, 0)),
            pl.BlockSpec((1, D_FEAT), lambda i: (0, 0)),
        ],
        out_specs=pl.BlockSpec((MM_BLK, D_FEAT), lambda i: (i, 0)),
        out_shape=jax.ShapeDtypeStruct((N_NODES, D_FEAT), jnp.float32),
    )(parts_a, parts_ct, W, b2d)


def kernel(x, edge_index, W, b):
    ei = edge_index.astype(jnp.int32)
    zeros_a = jnp.zeros((N_NODES, D_FEAT), jnp.float32)
    zeros_c = jnp.zeros((N_NODES,), jnp.float32)
    parts_a, parts_c = _sc_aggregate(ei, x, zeros_a, zeros_c)
    parts_ct = parts_c.reshape(NC * NS, N_NODES).T
    return _tc_finish(parts_a, parts_ct, W, b.reshape(1, D_FEAT))


# speedup vs baseline: 1.0671x; 1.0671x over previous
import jax
import jax.numpy as jnp
from jax import lax
from jax.experimental import pallas as pl
from jax.experimental.pallas import tpu as pltpu
from jax.experimental.pallas import tpu_sc as plsc

N_NODES = 10000
N_EDGES = 320000
D_FEAT = 128

NC = 2
NS = 16
NW = NC * NS
EDGES_PER_W = N_EDGES // NW
CHUNK = 128
NCHUNK = EDGES_PER_W // CHUNK
TAIL = EDGES_PER_W - NCHUNK * CHUNK
NBLK = 13
BLKCH = NCHUNK // NBLK
BLKE = BLKCH * CHUNK
ROWS_PER_TILE = N_NODES // NS


def _sc_body(ei_hbm, x_hbm, za_hbm, zc_hbm, outa_hbm, outc_hbm,
             src_a, src_b, dst_a, dst_b, src_t, dst_t, rows0, rows1, rows_t,
             c_tile, a_sh, sem0, sem1, semi, semt):
    c = lax.axis_index("c")
    s = lax.axis_index("s")
    wid = s * NC + c
    lo = s * ROWS_PER_TILE
    ebase = wid * EDGES_PER_W

    init_copies = [
        pltpu.async_copy(za_hbm.at[pl.ds(lo, ROWS_PER_TILE)],
                         a_sh.at[pl.ds(lo, ROWS_PER_TILE)], semi),
        pltpu.async_copy(zc_hbm, c_tile, semi),
        pltpu.async_copy(ei_hbm.at[0].at[pl.ds(ebase, BLKE)], src_a, semi),
        pltpu.async_copy(ei_hbm.at[1].at[pl.ds(ebase, BLKE)], dst_a, semi),
        pltpu.async_copy(ei_hbm.at[0].at[pl.ds(ebase + NCHUNK * CHUNK, TAIL)],
                         src_t, semi),
        pltpu.async_copy(ei_hbm.at[1].at[pl.ds(ebase + NCHUNK * CHUNK, TAIL)],
                         dst_t, semi),
    ]
    for cp in init_copies:
        cp.wait()
    plsc.subcore_barrier()
    pltpu.async_copy(x_hbm.at[src_t], rows_t, semt)

    ones16 = jnp.ones((16,), jnp.float32)

    def gather(sv, j, rows, sem):
        return pltpu.async_copy(
            x_hbm.at[sv.at[pl.ds(j * CHUNK, CHUNK)]], rows, sem)

    def drain(rows, sem):
        pltpu.make_async_copy(x_hbm.at[src_a.at[pl.ds(0, CHUNK)]], rows,
                              sem).wait()

    def counts(dv, j):
        for k in range(CHUNK // 16):
            idx = dv[pl.ds(j * CHUNK + k * 16, 16)]
            plsc.addupdate_scatter(c_tile, [idx], ones16)

    def scatter(dv, j, rows):
        pltpu.sync_copy(rows, a_sh.at[dv.at[pl.ds(j * CHUNK, CHUNK)]],
                        add=True)

    idx_bufs = (src_a, dst_a), (src_b, dst_b)
    for blk in range(NBLK):
        sv, dv = idx_bufs[blk % 2]
        nsv, ndv = idx_bufs[(blk + 1) % 2]
        if blk + 1 < NBLK:
            nbase = ebase + (blk + 1) * BLKE
            pltpu.async_copy(ei_hbm.at[0].at[pl.ds(nbase, BLKE)], nsv, semi)
            pltpu.async_copy(ei_hbm.at[1].at[pl.ds(nbase, BLKE)], ndv, semi)

        gather(sv, 0, rows0, sem0)

        def pair_body(i, carry, sv=sv, dv=dv):
            j = 2 * i
            gather(sv, j + 1, rows1, sem1)
            counts(dv, j)
            drain(rows0, sem0)
            scatter(dv, j, rows0)
            gather(sv, j + 2, rows0, sem0)
            counts(dv, j + 1)
            drain(rows1, sem1)
            scatter(dv, j + 1, rows1)
            return carry

        lax.fori_loop(0, BLKCH // 2 - 1, pair_body, 0)
        gather(sv, BLKCH - 1, rows1, sem1)
        counts(dv, BLKCH - 2)
        drain(rows0, sem0)
        scatter(dv, BLKCH - 2, rows0)
        counts(dv, BLKCH - 1)
        drain(rows1, sem1)
        scatter(dv, BLKCH - 1, rows1)

        if blk + 1 < NBLK:
            pltpu.make_async_copy(ei_hbm.at[0].at[pl.ds(0, BLKE)], nsv,
                                  semi).wait()
            pltpu.make_async_copy(ei_hbm.at[1].at[pl.ds(0, BLKE)], ndv,
                                  semi).wait()

    plsc.addupdate_scatter(c_tile, [dst_t[...]], ones16)
    pltpu.make_async_copy(x_hbm.at[src_t], rows_t, semt).wait()
    pltpu.sync_copy(rows_t, a_sh.at[dst_t], add=True)
    plsc.subcore_barrier()

    out_copies = [
        pltpu.async_copy(a_sh.at[pl.ds(lo, ROWS_PER_TILE)],
                         outa_hbm.at[c].at[pl.ds(lo, ROWS_PER_TILE)], semi),
        pltpu.async_copy(c_tile, outc_hbm.at[c].at[s], semi),
    ]
    for cp in out_copies:
        cp.wait()


def _sc_aggregate(edge_index, x, zeros_a, zeros_c):
    mesh = plsc.VectorSubcoreMesh(core_axis_name="c", subcore_axis_name="s",
                                  num_cores=NC, num_subcores=NS)
    return pl.kernel(
        _sc_body,
        out_type=(
            jax.ShapeDtypeStruct((NC, N_NODES, D_FEAT), jnp.float32),
            jax.ShapeDtypeStruct((NC, NS, N_NODES), jnp.float32),
        ),
        mesh=mesh,
        compiler_params=pltpu.CompilerParams(use_tc_tiling_on_sc=False,
                                             needs_layout_passes=False),
        scratch_types=[
            pltpu.VMEM((BLKE,), jnp.int32),
            pltpu.VMEM((BLKE,), jnp.int32),
            pltpu.VMEM((BLKE,), jnp.int32),
            pltpu.VMEM((BLKE,), jnp.int32),
            pltpu.VMEM((TAIL,), jnp.int32),
            pltpu.VMEM((TAIL,), jnp.int32),
            pltpu.VMEM((CHUNK, D_FEAT), jnp.float32),
            pltpu.VMEM((CHUNK, D_FEAT), jnp.float32),
            pltpu.VMEM((TAIL, D_FEAT), jnp.float32),
            pltpu.VMEM((N_NODES,), jnp.float32),
            pltpu.VMEM_SHARED((N_NODES, D_FEAT), jnp.float32),
            pltpu.SemaphoreType.DMA,
            pltpu.SemaphoreType.DMA,
            pltpu.SemaphoreType.DMA,
            pltpu.SemaphoreType.DMA,
        ],
    )(edge_index, x, zeros_a, zeros_c)


def _mm_body(a_ref, c_ref, w_ref, b_ref, o_ref):
    a = a_ref[0] + a_ref[1]
    cnt = jnp.sum(c_ref[...].reshape(NC * NS, N_NODES), axis=0)
    o_ref[...] = (jnp.dot(a, w_ref[...], preferred_element_type=jnp.float32)
                  + cnt[:, None] * b_ref[...])


def _tc_finish(parts_a, parts_c, W, b2d):
    return pl.pallas_call(
        _mm_body,
        out_shape=jax.ShapeDtypeStruct((N_NODES, D_FEAT), jnp.float32),
    )(parts_a, parts_c, W, b2d)


def kernel(x, edge_index, W, b):
    ei = edge_index.astype(jnp.int32)
    zeros_a = jnp.zeros((N_NODES, D_FEAT), jnp.float32)
    zeros_c = jnp.zeros((N_NODES,), jnp.float32)
    parts_a, parts_c = _sc_aggregate(ei, x, zeros_a, zeros_c)
    return _tc_finish(parts_a, parts_c, W, b.reshape(1, D_FEAT))


# speedup vs baseline: 1.1290x; 1.0580x over previous
import jax
import jax.numpy as jnp
from jax import lax
from jax.experimental import pallas as pl
from jax.experimental.pallas import tpu as pltpu
from jax.experimental.pallas import tpu_sc as plsc

N_NODES = 10000
N_EDGES = 320000
D_FEAT = 128

NC = 2
NS = 16
NW = NC * NS
EDGES_PER_W = N_EDGES // NW
CHUNK = 128
NCHUNK = EDGES_PER_W // CHUNK
TAIL = EDGES_PER_W - NCHUNK * CHUNK
NBLK = 6
BLKCH = NCHUNK // NBLK
BLKE = BLKCH * CHUNK
ROWS_PER_TILE = N_NODES // NS


def _sc_body(ei_hbm, x_hbm, za_hbm, zc_hbm, outa_hbm, outc_hbm,
             src_a, src_b, dst_a, dst_b, src_t, dst_t, rows0, rows1,
             c_tile, a_sh, sem0, sem1, semi, semt):
    c = lax.axis_index("c")
    s = lax.axis_index("s")
    wid = s * NC + c
    lo = s * ROWS_PER_TILE
    ebase = wid * EDGES_PER_W

    init_copies = [
        pltpu.async_copy(za_hbm.at[pl.ds(lo, ROWS_PER_TILE)],
                         a_sh.at[pl.ds(lo, ROWS_PER_TILE)], semi),
        pltpu.async_copy(zc_hbm, c_tile, semi),
        pltpu.async_copy(ei_hbm.at[0].at[pl.ds(ebase, BLKE)], src_a, semi),
        pltpu.async_copy(ei_hbm.at[1].at[pl.ds(ebase, BLKE)], dst_a, semi),
        pltpu.async_copy(ei_hbm.at[0].at[pl.ds(ebase + NCHUNK * CHUNK, TAIL)],
                         src_t, semi),
        pltpu.async_copy(ei_hbm.at[1].at[pl.ds(ebase + NCHUNK * CHUNK, TAIL)],
                         dst_t, semi),
    ]
    for cp in init_copies:
        cp.wait()
    plsc.subcore_barrier()

    ones16 = jnp.ones((16,), jnp.float32)

    def gather(sv, j, rows, sem):
        return pltpu.async_copy(
            x_hbm.at[sv.at[pl.ds(j * CHUNK, CHUNK)]], rows, sem)

    def drain(rows, sem):
        pltpu.make_async_copy(x_hbm.at[src_a.at[pl.ds(0, CHUNK)]], rows,
                              sem).wait()

    def counts(dv, j):
        for k in range(CHUNK // 16):
            idx = dv[pl.ds(j * CHUNK + k * 16, 16)]
            plsc.addupdate_scatter(c_tile, [idx], ones16)

    def scatter(dv, j, rows):
        pltpu.sync_copy(rows, a_sh.at[dv.at[pl.ds(j * CHUNK, CHUNK)]],
                        add=True)

    idx_bufs = (src_a, dst_a), (src_b, dst_b)
    for blk in range(NBLK):
        sv, dv = idx_bufs[blk % 2]
        nsv, ndv = idx_bufs[(blk + 1) % 2]
        if blk + 1 < NBLK:
            nbase = ebase + (blk + 1) * BLKE
            pltpu.async_copy(ei_hbm.at[0].at[pl.ds(nbase, BLKE)], nsv, semi)
            pltpu.async_copy(ei_hbm.at[1].at[pl.ds(nbase, BLKE)], ndv, semi)

        gather(sv, 0, rows0, sem0)

        def pair_body(i, carry, sv=sv, dv=dv):
            j = 2 * i
            gather(sv, j + 1, rows1, sem1)
            counts(dv, j)
            drain(rows0, sem0)
            scatter(dv, j, rows0)
            gather(sv, j + 2, rows0, sem0)
            counts(dv, j + 1)
            drain(rows1, sem1)
            scatter(dv, j + 1, rows1)
            return carry

        if BLKCH % 2 == 1:
            lax.fori_loop(0, (BLKCH - 1) // 2, pair_body, 0)
            counts(dv, BLKCH - 1)
            drain(rows0, sem0)
            scatter(dv, BLKCH - 1, rows0)
        else:
            lax.fori_loop(0, BLKCH // 2 - 1, pair_body, 0)
            gather(sv, BLKCH - 1, rows1, sem1)
            counts(dv, BLKCH - 2)
            drain(rows0, sem0)
            scatter(dv, BLKCH - 2, rows0)
            counts(dv, BLKCH - 1)
            drain(rows1, sem1)
            scatter(dv, BLKCH - 1, rows1)

        if blk + 1 < NBLK:
            pltpu.make_async_copy(ei_hbm.at[0].at[pl.ds(0, BLKE)], nsv,
                                  semi).wait()
            pltpu.make_async_copy(ei_hbm.at[1].at[pl.ds(0, BLKE)], ndv,
                                  semi).wait()

    pltpu.async_copy(x_hbm.at[src_t], rows0.at[pl.ds(0, TAIL)], semt)
    plsc.addupdate_scatter(c_tile, [dst_t[...]], ones16)
    pltpu.make_async_copy(x_hbm.at[src_t], rows0.at[pl.ds(0, TAIL)],
                          semt).wait()
    pltpu.sync_copy(rows0.at[pl.ds(0, TAIL)], a_sh.at[dst_t], add=True)
    plsc.subcore_barrier()

    out_copies = [
        pltpu.async_copy(a_sh.at[pl.ds(lo, ROWS_PER_TILE)],
                         outa_hbm.at[c].at[pl.ds(lo, ROWS_PER_TILE)], semi),
        pltpu.async_copy(c_tile, outc_hbm.at[c].at[s], semi),
    ]
    for cp in out_copies:
        cp.wait()


def _sc_aggregate(edge_index, x, zeros_a, zeros_c):
    mesh = plsc.VectorSubcoreMesh(core_axis_name="c", subcore_axis_name="s",
                                  num_cores=NC, num_subcores=NS)
    return pl.kernel(
        _sc_body,
        out_type=(
            jax.ShapeDtypeStruct((NC, N_NODES, D_FEAT), jnp.float32),
            jax.ShapeDtypeStruct((NC, NS, N_NODES), jnp.float32),
        ),
        mesh=mesh,
        compiler_params=pltpu.CompilerParams(use_tc_tiling_on_sc=False,
                                             needs_layout_passes=False),
        scratch_types=[
            pltpu.VMEM((BLKE,), jnp.int32),
            pltpu.VMEM((BLKE,), jnp.int32),
            pltpu.VMEM((BLKE,), jnp.int32),
            pltpu.VMEM((BLKE,), jnp.int32),
            pltpu.VMEM((TAIL,), jnp.int32),
            pltpu.VMEM((TAIL,), jnp.int32),
            pltpu.VMEM((CHUNK, D_FEAT), jnp.float32),
            pltpu.VMEM((CHUNK, D_FEAT), jnp.float32),
            pltpu.VMEM((N_NODES,), jnp.float32),
            pltpu.VMEM_SHARED((N_NODES, D_FEAT), jnp.float32),
            pltpu.SemaphoreType.DMA,
            pltpu.SemaphoreType.DMA,
            pltpu.SemaphoreType.DMA,
            pltpu.SemaphoreType.DMA,
        ],
    )(edge_index, x, zeros_a, zeros_c)


def _mm_body(a_ref, c_ref, w_ref, b_ref, o_ref):
    a = a_ref[0] + a_ref[1]
    cnt = jnp.sum(c_ref[...].reshape(NC * NS, N_NODES), axis=0)
    o_ref[...] = (jnp.dot(a, w_ref[...], preferred_element_type=jnp.float32)
                  + cnt[:, None] * b_ref[...])


def _tc_finish(parts_a, parts_c, W, b2d):
    return pl.pallas_call(
        _mm_body,
        out_shape=jax.ShapeDtypeStruct((N_NODES, D_FEAT), jnp.float32),
    )(parts_a, parts_c, W, b2d)


def kernel(x, edge_index, W, b):
    ei = edge_index.astype(jnp.int32)
    zeros_a = jnp.zeros((N_NODES, D_FEAT), jnp.float32)
    zeros_c = jnp.zeros((N_NODES,), jnp.float32)
    parts_a, parts_c = _sc_aggregate(ei, x, zeros_a, zeros_c)
    return _tc_finish(parts_a, parts_c, W, b.reshape(1, D_FEAT))


# speedup vs baseline: 1.1945x; 1.0580x over previous
import jax
import jax.numpy as jnp
from jax import lax
from jax.experimental import pallas as pl
from jax.experimental.pallas import tpu as pltpu
from jax.experimental.pallas import tpu_sc as plsc

N_NODES = 10000
N_EDGES = 320000
D_FEAT = 128

NC = 2
NS = 16
NW = NC * NS
EDGES_PER_W = N_EDGES // NW
CHUNK = 128
NCHUNK = EDGES_PER_W // CHUNK
TAIL = EDGES_PER_W - NCHUNK * CHUNK
NBLK = 6
BLKCH = NCHUNK // NBLK
BLKE = BLKCH * CHUNK
ROWS_PER_TILE = N_NODES // NS


def _sc_body(ei_hbm, x_hbm, za_hbm, zc_hbm, outa_hbm, outc_hbm,
             src_a, src_b, dst_a, dst_b, src_t, dst_t, rows0, rows1,
             c_tile, a_sh, sem0, sem1, semi, semt):
    c = lax.axis_index("c")
    s = lax.axis_index("s")
    wid = s * NC + c
    lo = s * ROWS_PER_TILE
    ebase = wid * EDGES_PER_W

    init_copies = [
        pltpu.async_copy(za_hbm.at[pl.ds(lo, ROWS_PER_TILE)],
                         a_sh.at[pl.ds(lo, ROWS_PER_TILE)], semi),
        pltpu.async_copy(zc_hbm, c_tile, semi),
        pltpu.async_copy(ei_hbm.at[0].at[pl.ds(ebase, BLKE)], src_a, semi),
        pltpu.async_copy(ei_hbm.at[1].at[pl.ds(ebase, BLKE)], dst_a, semi),
        pltpu.async_copy(ei_hbm.at[0].at[pl.ds(ebase + NCHUNK * CHUNK, TAIL)],
                         src_t, semi),
        pltpu.async_copy(ei_hbm.at[1].at[pl.ds(ebase + NCHUNK * CHUNK, TAIL)],
                         dst_t, semi),
    ]
    for cp in init_copies:
        cp.wait()
    plsc.subcore_barrier()

    ones16 = jnp.ones((16,), jnp.float32)

    def gather(sv, j, rows, sem):
        return pltpu.async_copy(
            x_hbm.at[sv.at[pl.ds(j * CHUNK, CHUNK)]], rows, sem)

    def drain(rows, sem):
        pltpu.make_async_copy(x_hbm.at[src_a.at[pl.ds(0, CHUNK)]], rows,
                              sem).wait()

    def counts(dv, j):
        for k in range(CHUNK // 16):
            idx = dv[pl.ds(j * CHUNK + k * 16, 16)]
            plsc.addupdate_scatter(c_tile, [idx], ones16)

    def scatter(dv, j, rows):
        pltpu.sync_copy(rows, a_sh.at[dv.at[pl.ds(j * CHUNK, CHUNK)]],
                        add=True)

    assert BLKCH % 2 == 1
    idx_bufs = (src_a, dst_a), (src_b, dst_b)
    row_bufs = (rows0, sem0, rows1, sem1), (rows1, sem1, rows0, sem0)
    gather(src_a, 0, rows0, sem0)
    for blk in range(NBLK):
        sv, dv = idx_bufs[blk % 2]
        nsv, ndv = idx_bufs[(blk + 1) % 2]
        ra, sa, rb, sb = row_bufs[blk % 2]
        if blk + 1 < NBLK:
            nbase = ebase + (blk + 1) * BLKE
            pltpu.async_copy(ei_hbm.at[0].at[pl.ds(nbase, BLKE)], nsv, semi)
            pltpu.async_copy(ei_hbm.at[1].at[pl.ds(nbase, BLKE)], ndv, semi)

        def pair_body(i, carry, sv=sv, dv=dv, ra=ra, sa=sa, rb=rb, sb=sb):
            j = 2 * i
            gather(sv, j + 1, rb, sb)
            counts(dv, j)
            drain(ra, sa)
            scatter(dv, j, ra)
            gather(sv, j + 2, ra, sa)
            counts(dv, j + 1)
            drain(rb, sb)
            scatter(dv, j + 1, rb)
            return carry

        lax.fori_loop(0, (BLKCH - 1) // 2, pair_body, 0)
        if blk + 1 < NBLK:
            pltpu.make_async_copy(ei_hbm.at[0].at[pl.ds(0, BLKE)], nsv,
                                  semi).wait()
            pltpu.make_async_copy(ei_hbm.at[1].at[pl.ds(0, BLKE)], ndv,
                                  semi).wait()
            gather(nsv, 0, rb, sb)
        counts(dv, BLKCH - 1)
        drain(ra, sa)
        scatter(dv, BLKCH - 1, ra)

    pltpu.async_copy(x_hbm.at[src_t], rows0.at[pl.ds(0, TAIL)], semt)
    plsc.addupdate_scatter(c_tile, [dst_t[...]], ones16)
    pltpu.make_async_copy(x_hbm.at[src_t], rows0.at[pl.ds(0, TAIL)],
                          semt).wait()
    pltpu.sync_copy(rows0.at[pl.ds(0, TAIL)], a_sh.at[dst_t], add=True)
    plsc.subcore_barrier()

    out_copies = [
        pltpu.async_copy(a_sh.at[pl.ds(lo, ROWS_PER_TILE)],
                         outa_hbm.at[c].at[pl.ds(lo, ROWS_PER_TILE)], semi),
        pltpu.async_copy(c_tile, outc_hbm.at[c].at[s], semi),
    ]
    for cp in out_copies:
        cp.wait()


def _sc_aggregate(edge_index, x, zeros_a, zeros_c):
    mesh = plsc.VectorSubcoreMesh(core_axis_name="c", subcore_axis_name="s",
                                  num_cores=NC, num_subcores=NS)
    return pl.kernel(
        _sc_body,
        out_type=(
            jax.ShapeDtypeStruct((NC, N_NODES, D_FEAT), jnp.float32),
            jax.ShapeDtypeStruct((NC, NS, N_NODES), jnp.float32),
        ),
        mesh=mesh,
        compiler_params=pltpu.CompilerParams(use_tc_tiling_on_sc=False,
                                             needs_layout_passes=False),
        scratch_types=[
            pltpu.VMEM((BLKE,), jnp.int32),
            pltpu.VMEM((BLKE,), jnp.int32),
            pltpu.VMEM((BLKE,), jnp.int32),
            pltpu.VMEM((BLKE,), jnp.int32),
            pltpu.VMEM((TAIL,), jnp.int32),
            pltpu.VMEM((TAIL,), jnp.int32),
            pltpu.VMEM((CHUNK, D_FEAT), jnp.float32),
            pltpu.VMEM((CHUNK, D_FEAT), jnp.float32),
            pltpu.VMEM((N_NODES,), jnp.float32),
            pltpu.VMEM_SHARED((N_NODES, D_FEAT), jnp.float32),
            pltpu.SemaphoreType.DMA,
            pltpu.SemaphoreType.DMA,
            pltpu.SemaphoreType.DMA,
            pltpu.SemaphoreType.DMA,
        ],
    )(edge_index, x, zeros_a, zeros_c)


def _mm_body(a_ref, c_ref, w_ref, b_ref, o_ref):
    a = a_ref[0] + a_ref[1]
    cnt = jnp.sum(c_ref[...].reshape(NC * NS, N_NODES), axis=0)
    o_ref[...] = (jnp.dot(a, w_ref[...], preferred_element_type=jnp.float32)
                  + cnt[:, None] * b_ref[...])


def _tc_finish(parts_a, parts_c, W, b2d):
    return pl.pallas_call(
        _mm_body,
        out_shape=jax.ShapeDtypeStruct((N_NODES, D_FEAT), jnp.float32),
    )(parts_a, parts_c, W, b2d)


def kernel(x, edge_index, W, b):
    ei = edge_index.astype(jnp.int32)
    zeros_a = jnp.zeros((N_NODES, D_FEAT), jnp.float32)
    zeros_c = jnp.zeros((N_NODES,), jnp.float32)
    parts_a, parts_c = _sc_aggregate(ei, x, zeros_a, zeros_c)
    return _tc_finish(parts_a, parts_c, W, b.reshape(1, D_FEAT))
